# baseline (speedup 1.0000x reference)
import jax
import jax.numpy as jnp
from jax import lax
from jax.experimental import pallas as pl
from jax.experimental.pallas import tpu as pltpu
from jax.experimental.pallas import tpu_sc as plsc

_N_NODES = 10000
_N_EDGES = 320000
_NF = 128
_NH = 64
_NHH = 32
_NT = 128

_NC = 2
_NS = 16
_NW = _NC * _NS
_NPIPE = 4
_NE_H = _N_EDGES // _NPIPE
_CHUNK = 80
_NCHT = _NE_H // _CHUNK
_JG = (_NCHT + _NW - 1) // _NW
_CNTW = 16

_SPAN = 624
_TAIL = _N_NODES - _NS * _SPAN
_TAIL_BASE = _NS * _SPAN


def _sc_mesh():
    return plsc.VectorSubcoreMesh(core_axis_name="c", subcore_axis_name="s")


def _sc_gather(xa, col3d, row3d):
    def body(xa_hbm, col_hbm, row_hbm, g_hbm, pc_hbm,
             cidx_v, ridx_v, rows_v, ones_v, cwb_v, cnt_s, sem):
        sid = lax.axis_index("s")
        cid = lax.axis_index("c")
        wid = sid * _NC + cid
        nbase = sid * _SPAN

        def zfill(i, carry):
            cwb_v[i, :] = jnp.zeros((_CNTW,), jnp.float32)
            return carry

        lax.fori_loop(0, _SPAN, zfill, 0)

        def ofill(i, carry):
            ones_v[i, :] = jnp.ones((_CNTW,), jnp.float32)
            return carry

        lax.fori_loop(0, _CHUNK, ofill, 0)

        pltpu.sync_copy(cwb_v, cnt_s.at[pl.ds(nbase, _SPAN)])

        @pl.when(sid == _NS - 1)
        def _tail_zero():
            pltpu.sync_copy(cwb_v.at[pl.ds(0, _TAIL)],
                            cnt_s.at[pl.ds(_TAIL_BASE, _TAIL)])

        pltpu.sync_copy(col_hbm.at[wid], cidx_v)
        pltpu.sync_copy(row_hbm.at[wid], ridx_v)
        plsc.subcore_barrier()

        def step(j, carry):
            c = j * _NW + wid

            @pl.when(c < _NCHT)
            def _do():
                pltpu.async_copy(xa_hbm.at[cidx_v.at[j]], rows_v, sem).wait()
                pltpu.sync_copy(rows_v, g_hbm.at[pl.ds(c * _CHUNK, _CHUNK)])
                pltpu.sync_copy(ones_v, cnt_s.at[ridx_v.at[j]], add=True)

            return carry

        lax.fori_loop(0, _JG, step, 0)
        plsc.subcore_barrier()

        obase = cid * _N_NODES + nbase
        pltpu.sync_copy(cnt_s.at[pl.ds(nbase, _SPAN)], cwb_v)
        pltpu.sync_copy(cwb_v, pc_hbm.at[pl.ds(obase, _SPAN)])

        @pl.when(sid == _NS - 1)
        def _tail_wb():
            pltpu.sync_copy(cnt_s.at[pl.ds(_TAIL_BASE, _TAIL)],
                            cwb_v.at[pl.ds(0, _TAIL)])
            pltpu.sync_copy(cwb_v.at[pl.ds(0, _TAIL)],
                            pc_hbm.at[pl.ds(cid * _N_NODES + _TAIL_BASE,
                                            _TAIL)])

    f = pl.kernel(
        body,
        out_type=(
            jax.ShapeDtypeStruct((_NE_H, _NF), jnp.float32),
            jax.ShapeDtypeStruct((_NC * _N_NODES, _CNTW), jnp.float32),
        ),
        mesh=_sc_mesh(),
        compiler_params=pltpu.CompilerParams(use_tc_tiling_on_sc=False),
        scratch_types=[
            pltpu.VMEM((_JG, _CHUNK), jnp.int32),
            pltpu.VMEM((_JG, _CHUNK), jnp.int32),
            pltpu.VMEM((_CHUNK, _NF), jnp.float32),
            pltpu.VMEM((_CHUNK, _CNTW), jnp.float32),
            pltpu.VMEM((_SPAN, _CNTW), jnp.float32),
            pltpu.VMEM_SHARED((_N_NODES, _CNTW), jnp.float32),
            pltpu.SemaphoreType.DMA,
        ],
    )
    return f(xa, col3d, row3d)


_RCH = 40
_SP = 208
_NSP = _SPAN // _SP


def _sc_scatter(hp, bidx):
    def body(hp_hbm, bidx_hbm, ps_hbm, idx_v, ha_v, hb_v, zbuf, obuf, acc):
        sid = lax.axis_index("s")
        cid = lax.axis_index("c")
        wid = sid * _NC + cid
        nbase = sid * _SPAN

        def zfill(t, carry):
            zbuf[t // 4, pl.ds((t % 4) * 16, 16)] = jnp.zeros((16,),
                                                              jnp.float32)
            return carry

        lax.fori_loop(0, _SP * 4, zfill, 0)
        pltpu.sync_copy(bidx_hbm.at[wid], idx_v)

        for s in range(_NSP):
            pltpu.sync_copy(zbuf, acc.at[pl.ds(nbase + s * _SP, _SP)])

        @pl.when(sid == _NS - 1)
        def _tail_zero():
            pltpu.sync_copy(zbuf.at[pl.ds(0, _TAIL)],
                            acc.at[pl.ds(_TAIL_BASE, _TAIL)])

        plsc.subcore_barrier()

        def step(j, carry):
            c = j * _NW + wid

            @pl.when(c < _NCHT)
            def _do():
                pltpu.sync_copy(
                    hp_hbm.at[pl.ds(c * _RCH, _RCH), pl.ds(0, _NH)],
                    ha_v)
                pltpu.sync_copy(
                    hp_hbm.at[pl.ds(c * _RCH, _RCH), pl.ds(_NH, _NH)],
                    hb_v)
                pltpu.sync_copy(ha_v, acc.at[idx_v.at[2 * j]], add=True)
                pltpu.sync_copy(hb_v, acc.at[idx_v.at[2 * j + 1]], add=True)

            return carry

        lax.fori_loop(0, _JG, step, 0)
        plsc.subcore_barrier()

        obase = cid * _N_NODES + nbase
        for s in range(_NSP):
            pltpu.sync_copy(acc.at[pl.ds(nbase + s * _SP, _SP)], obuf)
            pltpu.sync_copy(obuf, ps_hbm.at[pl.ds(obase + s * _SP, _SP)])

        @pl.when(sid == _NS - 1)
        def _tail_wb():
            pltpu.sync_copy(acc.at[pl.ds(_TAIL_BASE, _TAIL)],
                            obuf.at[pl.ds(0, _TAIL)])
            pltpu.sync_copy(
                obuf.at[pl.ds(0, _TAIL)],
                ps_hbm.at[pl.ds(cid * _N_NODES + _TAIL_BASE, _TAIL)])

    f = pl.kernel(
        body,
        out_type=jax.ShapeDtypeStruct((2 * _N_NODES, _NH), jnp.float32),
        mesh=_sc_mesh(),
        compiler_params=pltpu.CompilerParams(use_tc_tiling_on_sc=False),
        scratch_types=[
            pltpu.VMEM((2 * _JG, _RCH), jnp.int32),
            pltpu.VMEM((_RCH, _NH), jnp.float32),
            pltpu.VMEM((_RCH, _NH), jnp.float32),
            pltpu.VMEM((_SP, _NH), jnp.float32),
            pltpu.VMEM((_SP, _NH), jnp.float32),
            pltpu.VMEM_SHARED((_N_NODES, _NH), jnp.float32),
        ],
    )
    return f(hp, bidx)


def _tc_xa(x, w_top_pad):
    def body(x_ref, w_ref, o_ref):
        o_ref[...] = jnp.dot(x_ref[...], w_ref[...],
                             preferred_element_type=jnp.float32)

    return pl.pallas_call(
        body,
        grid=(10,),
        in_specs=[
            pl.BlockSpec((_N_NODES // 10, _NF), lambda i: (i, 0)),
            pl.BlockSpec((_NF, _NF), lambda i: (0, 0)),
        ],
        out_specs=pl.BlockSpec((_N_NODES // 10, _NF), lambda i: (i, 0)),
        out_shape=jax.ShapeDtypeStruct((_N_NODES, _NF), jnp.float32),
    )(x, w_top_pad)


_EB = 2000


def _tc_edge(g, ea, w_bot, b1a, g1, be1, w2, b2, hoff):
    def body(g_ref, ea_ref, wa_ref, ba_ref, g1_ref, be_ref, wb_ref, bb_ref,
             o_ref):
        t = jnp.dot(ea_ref[...], wa_ref[...],
                    preferred_element_type=jnp.float32)
        t = t + g_ref[...][:, :_NH] + ba_ref[...]
        t = jnp.maximum(t, 0.0)
        m = jnp.mean(t, axis=-1, keepdims=True)
        v = jnp.mean((t - m) * (t - m), axis=-1, keepdims=True)
        t = (t - m) * lax.rsqrt(v + 1e-5) * g1_ref[...] + be_ref[...]
        h = jnp.dot(t, wb_ref[...],
                    preferred_element_type=jnp.float32) + bb_ref[...]
        o_ref[...] = jnp.concatenate([h[:_EB // 2], h[_EB // 2:]], axis=1)

    n_blocks = _NE_H // _EB
    vec = lambda: pl.BlockSpec((1, _NH), lambda i: (0, 0))
    mat = lambda: pl.BlockSpec((_NH, _NH), lambda i: (0, 0))
    return pl.pallas_call(
        body,
        grid=(n_blocks,),
        in_specs=[
            pl.BlockSpec((_EB, _NF), lambda i: (i, 0)),
            pl.BlockSpec((_EB, _NH), lambda i: (i + hoff, 0)),
            mat(), vec(), vec(), vec(), mat(), vec(),
        ],
        out_specs=pl.BlockSpec((_EB // 2, _NF), lambda i: (i, 0)),
        out_shape=jax.ShapeDtypeStruct((_NE_H // 2, _NF), jnp.float32),
    )(g, ea, w_bot, b1a, g1, be1, w2, b2)


def _tc_node(x, ps_parts, pc_parts, w_top, w_bot, b2a, g2,
             be2, w2b, b2b):
    nps = len(ps_parts)
    npc = len(pc_parts)

    def body(*refs):
        x_ref = refs[0]
        ps_refs = refs[1:1 + nps]
        pc_refs = refs[1 + nps:1 + nps + npc]
        (wt_ref, wb_ref, ba_ref, g2_ref, be_ref, wo_ref, bo_ref,
         o_ref) = refs[1 + nps + npc:]
        cnt = pc_refs[0][...][:, 0:1]
        for r in pc_refs[1:]:
            cnt = cnt + r[...][:, 0:1]
        s = ps_refs[0][...]
        for r in ps_refs[1:]:
            s = s + r[...]
        mean = s / jnp.maximum(cnt, 1.0)
        t = (jnp.dot(x_ref[...], wt_ref[...],
                     preferred_element_type=jnp.float32)
             + jnp.dot(mean, wb_ref[...], preferred_element_type=jnp.float32)
             + ba_ref[...])
        t = jnp.maximum(t, 0.0)
        m = jnp.mean(t, axis=-1, keepdims=True)
        v = jnp.mean((t - m) * (t - m), axis=-1, keepdims=True)
        t = (t - m) * lax.rsqrt(v + 1e-5) * g2_ref[...] + be_ref[...]
        o_ref[...] = jnp.dot(t, wo_ref[...],
                             preferred_element_type=jnp.float32) + bo_ref[...]

    nb = _N_NODES // 10
    vecH = lambda: pl.BlockSpec((1, _NH), lambda i: (0, 0))
    vecT = lambda: pl.BlockSpec((1, _NT), lambda i: (0, 0))
    return pl.pallas_call(
        body,
        grid=(10,),
        in_specs=(
            [pl.BlockSpec((nb, _NF), lambda i: (i, 0))]
            + [pl.BlockSpec((nb, _NH), lambda i: (i, 0))] * nps
            + [pl.BlockSpec((nb, _CNTW), lambda i: (i, 0))] * npc
            + [
                pl.BlockSpec((_NF, _NH), lambda i: (0, 0)),
                pl.BlockSpec((_NH, _NH), lambda i: (0, 0)),
                vecH(), vecH(), vecH(),
                pl.BlockSpec((_NH, _NT), lambda i: (0, 0)),
                vecT(),
            ]
        ),
        out_specs=pl.BlockSpec((nb, _NT), lambda i: (i, 0)),
        out_shape=jax.ShapeDtypeStruct((_N_NODES, _NT), jnp.float32),
    )(x, *ps_parts, *pc_parts, w_top, w_bot, b2a, g2, be2,
      w2b, b2b)


def _half_arrays(colh, rowh):
    npad = _JG * _NW - _NCHT
    c2 = jnp.pad(colh.reshape(_NCHT, _CHUNK), ((0, npad), (0, 0)))
    col3d = c2.reshape(_JG, _NW, _CHUNK).transpose(1, 0, 2)
    r2 = jnp.pad(rowh.reshape(_NCHT, _CHUNK), ((0, npad), (0, 0)))
    row3d = r2.reshape(_JG, _NW, _CHUNK).transpose(1, 0, 2)

    flat = rowh.reshape(_NE_H // _EB, 2, _EB // 2).transpose(0, 2, 1)
    b = jnp.pad(flat.reshape(_NCHT, _RCH, 2), ((0, npad), (0, 0), (0, 0)))
    bidx = (b.reshape(_JG, _NW, _RCH, 2)
            .transpose(1, 0, 3, 2)
            .reshape(_NW, 2 * _JG, _RCH))
    return col3d, row3d, bidx


def kernel(x, edge_idx, edge_attr, W1a, b1a, g1, be1, W1b, b1b,
           W2a, b2a, g2, be2, W2b, b2b):
    row = edge_idx[0].astype(jnp.int32)
    col = edge_idx[1].astype(jnp.int32)

    parts = [_half_arrays(col[h * _NE_H:(h + 1) * _NE_H],
                          row[h * _NE_H:(h + 1) * _NE_H])
             for h in range(_NPIPE)]

    w_top_pad = jnp.pad(W1a[:_NF], ((0, 0), (0, _NF - _NH)))
    xa = _tc_xa(x, w_top_pad)

    gs = [_sc_gather(xa, c3, r3) for (c3, r3, _) in parts]

    eargs = (W1a[_NF:], b1a.reshape(1, _NH), g1.reshape(1, _NH),
             be1.reshape(1, _NH), W1b, b1b.reshape(1, _NH))
    hps = [_tc_edge(g, edge_attr, *eargs, h * (_NE_H // _EB))
           for h, (g, _) in enumerate(gs)]

    pss = [_sc_scatter(hp, bi) for hp, (_, _, bi) in zip(hps, parts)]

    n = _N_NODES
    ps_parts = [p for ps in pss for p in (ps[:n], ps[n:])]
    pc_parts = [p for (_, pc) in gs for p in (pc[:n], pc[n:])]
    return _tc_node(x, ps_parts, pc_parts,
                    W2a[:_NF], W2a[_NF:],
                    b2a.reshape(1, _NH), g2.reshape(1, _NH),
                    be2.reshape(1, _NH), W2b, b2b.reshape(1, _NT))


# speedup vs baseline: 1.1261x; 1.1261x over previous
import jax
import jax.numpy as jnp
from jax import lax
from jax.experimental import pallas as pl
from jax.experimental.pallas import tpu as pltpu
from jax.experimental.pallas import tpu_sc as plsc

_N_NODES = 10000
_N_EDGES = 320000
_NF = 128
_NH = 64
_NHH = 32
_NT = 128

_NC = 2
_NS = 16
_NW = _NC * _NS
_NE_H = _N_EDGES // 2
_CHUNK = 80
_NCHT = _NE_H // _CHUNK
_JG = (_NCHT + _NW - 1) // _NW
_CNTW = 16

_SPAN = 624
_TAIL = _N_NODES - _NS * _SPAN
_TAIL_BASE = _NS * _SPAN


def _sc_mesh():
    return plsc.VectorSubcoreMesh(core_axis_name="c", subcore_axis_name="s")


def _sc_gather(xa, col3d, row3d):
    def body(xa_hbm, col_hbm, row_hbm, g_hbm, pc_hbm,
             cidx_v, ridx_v, rows_v, ones_v, cwb_v, cnt_s, sem):
        sid = lax.axis_index("s")
        cid = lax.axis_index("c")
        wid = sid * _NC + cid
        nbase = sid * _SPAN

        def zfill(i, carry):
            cwb_v[i, :] = jnp.zeros((_CNTW,), jnp.float32)
            return carry

        lax.fori_loop(0, _SPAN, zfill, 0)

        def ofill(i, carry):
            ones_v[i, :] = jnp.ones((_CNTW,), jnp.float32)
            return carry

        lax.fori_loop(0, _CHUNK, ofill, 0)

        pltpu.sync_copy(cwb_v, cnt_s.at[pl.ds(nbase, _SPAN)])

        @pl.when(sid == _NS - 1)
        def _tail_zero():
            pltpu.sync_copy(cwb_v.at[pl.ds(0, _TAIL)],
                            cnt_s.at[pl.ds(_TAIL_BASE, _TAIL)])

        pltpu.sync_copy(col_hbm.at[wid], cidx_v)
        pltpu.sync_copy(row_hbm.at[wid], ridx_v)
        plsc.subcore_barrier()

        def step(j, carry):
            c = j * _NW + wid

            @pl.when(c < _NCHT)
            def _do():
                pltpu.async_copy(xa_hbm.at[cidx_v.at[j]], rows_v, sem).wait()
                pltpu.sync_copy(rows_v, g_hbm.at[pl.ds(c * _CHUNK, _CHUNK)])
                pltpu.sync_copy(ones_v, cnt_s.at[ridx_v.at[j]], add=True)

            return carry

        lax.fori_loop(0, _JG, step, 0)
        plsc.subcore_barrier()

        obase = cid * _N_NODES + nbase
        pltpu.sync_copy(cnt_s.at[pl.ds(nbase, _SPAN)], cwb_v)
        pltpu.sync_copy(cwb_v, pc_hbm.at[pl.ds(obase, _SPAN)])

        @pl.when(sid == _NS - 1)
        def _tail_wb():
            pltpu.sync_copy(cnt_s.at[pl.ds(_TAIL_BASE, _TAIL)],
                            cwb_v.at[pl.ds(0, _TAIL)])
            pltpu.sync_copy(cwb_v.at[pl.ds(0, _TAIL)],
                            pc_hbm.at[pl.ds(cid * _N_NODES + _TAIL_BASE,
                                            _TAIL)])

    f = pl.kernel(
        body,
        out_type=(
            jax.ShapeDtypeStruct((_NE_H, _NF), jnp.float32),
            jax.ShapeDtypeStruct((_NC * _N_NODES, _CNTW), jnp.float32),
        ),
        mesh=_sc_mesh(),
        compiler_params=pltpu.CompilerParams(use_tc_tiling_on_sc=False),
        scratch_types=[
            pltpu.VMEM((_JG, _CHUNK), jnp.int32),
            pltpu.VMEM((_JG, _CHUNK), jnp.int32),
            pltpu.VMEM((_CHUNK, _NF), jnp.float32),
            pltpu.VMEM((_CHUNK, _CNTW), jnp.float32),
            pltpu.VMEM((_SPAN, _CNTW), jnp.float32),
            pltpu.VMEM_SHARED((_N_NODES, _CNTW), jnp.float32),
            pltpu.SemaphoreType.DMA,
        ],
    )
    return f(xa, col3d, row3d)


_RCH = 40
_SP = 208
_NSP = _SPAN // _SP


def _sc_scatter(hp, bidx):
    def body(hp_hbm, bidx_hbm, ps_hbm, idx_v, ha_v, hb_v, zbuf, obuf, acc):
        sid = lax.axis_index("s")
        cid = lax.axis_index("c")
        wid = sid * _NC + cid
        nbase = sid * _SPAN

        def zfill(t, carry):
            zbuf[t // 4, pl.ds((t % 4) * 16, 16)] = jnp.zeros((16,),
                                                              jnp.float32)
            return carry

        lax.fori_loop(0, _SP * 4, zfill, 0)
        pltpu.sync_copy(bidx_hbm.at[wid], idx_v)

        for s in range(_NSP):
            pltpu.sync_copy(zbuf, acc.at[pl.ds(nbase + s * _SP, _SP)])

        @pl.when(sid == _NS - 1)
        def _tail_zero():
            pltpu.sync_copy(zbuf.at[pl.ds(0, _TAIL)],
                            acc.at[pl.ds(_TAIL_BASE, _TAIL)])

        plsc.subcore_barrier()

        def step(j, carry):
            c = j * _NW + wid

            @pl.when(c < _NCHT)
            def _do():
                pltpu.sync_copy(
                    hp_hbm.at[pl.ds(c * _RCH, _RCH), pl.ds(0, _NH)],
                    ha_v)
                pltpu.sync_copy(
                    hp_hbm.at[pl.ds(c * _RCH, _RCH), pl.ds(_NH, _NH)],
                    hb_v)
                pltpu.sync_copy(ha_v, acc.at[idx_v.at[2 * j]], add=True)
                pltpu.sync_copy(hb_v, acc.at[idx_v.at[2 * j + 1]], add=True)

            return carry

        lax.fori_loop(0, _JG, step, 0)
        plsc.subcore_barrier()

        obase = cid * _N_NODES + nbase
        for s in range(_NSP):
            pltpu.sync_copy(acc.at[pl.ds(nbase + s * _SP, _SP)], obuf)
            pltpu.sync_copy(obuf, ps_hbm.at[pl.ds(obase + s * _SP, _SP)])

        @pl.when(sid == _NS - 1)
        def _tail_wb():
            pltpu.sync_copy(acc.at[pl.ds(_TAIL_BASE, _TAIL)],
                            obuf.at[pl.ds(0, _TAIL)])
            pltpu.sync_copy(
                obuf.at[pl.ds(0, _TAIL)],
                ps_hbm.at[pl.ds(cid * _N_NODES + _TAIL_BASE, _TAIL)])

    f = pl.kernel(
        body,
        out_type=jax.ShapeDtypeStruct((2 * _N_NODES, _NH), jnp.float32),
        mesh=_sc_mesh(),
        compiler_params=pltpu.CompilerParams(use_tc_tiling_on_sc=False),
        scratch_types=[
            pltpu.VMEM((2 * _JG, _RCH), jnp.int32),
            pltpu.VMEM((_RCH, _NH), jnp.float32),
            pltpu.VMEM((_RCH, _NH), jnp.float32),
            pltpu.VMEM((_SP, _NH), jnp.float32),
            pltpu.VMEM((_SP, _NH), jnp.float32),
            pltpu.VMEM_SHARED((_N_NODES, _NH), jnp.float32),
        ],
    )
    return f(hp, bidx)


def _tc_xa(x, w_top_pad):
    def body(x_ref, w_ref, o_ref):
        o_ref[...] = jnp.dot(x_ref[...], w_ref[...],
                             preferred_element_type=jnp.float32)

    return pl.pallas_call(
        body,
        grid=(10,),
        in_specs=[
            pl.BlockSpec((_N_NODES // 10, _NF), lambda i: (i, 0)),
            pl.BlockSpec((_NF, _NF), lambda i: (0, 0)),
        ],
        out_specs=pl.BlockSpec((_N_NODES // 10, _NF), lambda i: (i, 0)),
        out_shape=jax.ShapeDtypeStruct((_N_NODES, _NF), jnp.float32),
    )(x, w_top_pad)


_EB = 2000


def _tc_edge(g, ea, w_bot, b1a, g1, be1, w2, b2, hoff):
    def body(g_ref, ea_ref, wa_ref, ba_ref, g1_ref, be_ref, wb_ref, bb_ref,
             o_ref):
        t = jnp.dot(ea_ref[...], wa_ref[...],
                    preferred_element_type=jnp.float32)
        t = t + g_ref[...][:, :_NH] + ba_ref[...]
        t = jnp.maximum(t, 0.0)
        m = jnp.mean(t, axis=-1, keepdims=True)
        v = jnp.mean((t - m) * (t - m), axis=-1, keepdims=True)
        t = (t - m) * lax.rsqrt(v + 1e-5) * g1_ref[...] + be_ref[...]
        h = jnp.dot(t, wb_ref[...],
                    preferred_element_type=jnp.float32) + bb_ref[...]
        o_ref[...] = jnp.concatenate([h[:_EB // 2], h[_EB // 2:]], axis=1)

    n_blocks = _NE_H // _EB
    vec = lambda: pl.BlockSpec((1, _NH), lambda i: (0, 0))
    mat = lambda: pl.BlockSpec((_NH, _NH), lambda i: (0, 0))
    return pl.pallas_call(
        body,
        grid=(n_blocks,),
        in_specs=[
            pl.BlockSpec((_EB, _NF), lambda i: (i, 0)),
            pl.BlockSpec((_EB, _NH), lambda i: (i + hoff, 0)),
            mat(), vec(), vec(), vec(), mat(), vec(),
        ],
        out_specs=pl.BlockSpec((_EB // 2, _NF), lambda i: (i, 0)),
        out_shape=jax.ShapeDtypeStruct((_NE_H // 2, _NF), jnp.float32),
    )(g, ea, w_bot, b1a, g1, be1, w2, b2)


def _tc_node(x, pa0, pa1, pb0, pb1, c0, c1, d0, d1, w_top, w_bot, b2a, g2,
             be2, w2b, b2b):
    def body(x_ref, pa0_ref, pa1_ref, pb0_ref, pb1_ref, c0_ref, c1_ref,
             d0_ref, d1_ref, wt_ref, wb_ref, ba_ref, g2_ref, be_ref, wo_ref,
             bo_ref, o_ref):
        cnt = (c0_ref[...][:, 0:1] + c1_ref[...][:, 0:1]
               + d0_ref[...][:, 0:1] + d1_ref[...][:, 0:1])
        s = pa0_ref[...] + pa1_ref[...] + pb0_ref[...] + pb1_ref[...]
        mean = s / jnp.maximum(cnt, 1.0)
        t = (jnp.dot(x_ref[...], wt_ref[...],
                     preferred_element_type=jnp.float32)
             + jnp.dot(mean, wb_ref[...], preferred_element_type=jnp.float32)
             + ba_ref[...])
        t = jnp.maximum(t, 0.0)
        m = jnp.mean(t, axis=-1, keepdims=True)
        v = jnp.mean((t - m) * (t - m), axis=-1, keepdims=True)
        t = (t - m) * lax.rsqrt(v + 1e-5) * g2_ref[...] + be_ref[...]
        o_ref[...] = jnp.dot(t, wo_ref[...],
                             preferred_element_type=jnp.float32) + bo_ref[...]

    nb = _N_NODES // 10
    vecH = lambda: pl.BlockSpec((1, _NH), lambda i: (0, 0))
    vecT = lambda: pl.BlockSpec((1, _NT), lambda i: (0, 0))
    return pl.pallas_call(
        body,
        grid=(10,),
        in_specs=[
            pl.BlockSpec((nb, _NF), lambda i: (i, 0)),
            pl.BlockSpec((nb, _NH), lambda i: (i, 0)),
            pl.BlockSpec((nb, _NH), lambda i: (i, 0)),
            pl.BlockSpec((nb, _NH), lambda i: (i, 0)),
            pl.BlockSpec((nb, _NH), lambda i: (i, 0)),
            pl.BlockSpec((nb, _CNTW), lambda i: (i, 0)),
            pl.BlockSpec((nb, _CNTW), lambda i: (i, 0)),
            pl.BlockSpec((nb, _CNTW), lambda i: (i, 0)),
            pl.BlockSpec((nb, _CNTW), lambda i: (i, 0)),
            pl.BlockSpec((_NF, _NH), lambda i: (0, 0)),
            pl.BlockSpec((_NH, _NH), lambda i: (0, 0)),
            vecH(), vecH(), vecH(),
            pl.BlockSpec((_NH, _NT), lambda i: (0, 0)),
            vecT(),
        ],
        out_specs=pl.BlockSpec((nb, _NT), lambda i: (i, 0)),
        out_shape=jax.ShapeDtypeStruct((_N_NODES, _NT), jnp.float32),
    )(x, pa0, pa1, pb0, pb1, c0, c1, d0, d1, w_top, w_bot, b2a, g2, be2,
      w2b, b2b)


def _half_arrays(colh, rowh):
    npad = _JG * _NW - _NCHT
    c2 = jnp.pad(colh.reshape(_NCHT, _CHUNK), ((0, npad), (0, 0)))
    col3d = c2.reshape(_JG, _NW, _CHUNK).transpose(1, 0, 2)
    r2 = jnp.pad(rowh.reshape(_NCHT, _CHUNK), ((0, npad), (0, 0)))
    row3d = r2.reshape(_JG, _NW, _CHUNK).transpose(1, 0, 2)

    flat = rowh.reshape(_NE_H // _EB, 2, _EB // 2).transpose(0, 2, 1)
    b = jnp.pad(flat.reshape(_NCHT, _RCH, 2), ((0, npad), (0, 0), (0, 0)))
    bidx = (b.reshape(_JG, _NW, _RCH, 2)
            .transpose(1, 0, 3, 2)
            .reshape(_NW, 2 * _JG, _RCH))
    return col3d, row3d, bidx


def kernel(x, edge_idx, edge_attr, W1a, b1a, g1, be1, W1b, b1b,
           W2a, b2a, g2, be2, W2b, b2b):
    row = edge_idx[0].astype(jnp.int32)
    col = edge_idx[1].astype(jnp.int32)

    c3a, r3a, bia = _half_arrays(col[:_NE_H], row[:_NE_H])
    c3b, r3b, bib = _half_arrays(col[_NE_H:], row[_NE_H:])

    w_top_pad = jnp.pad(W1a[:_NF], ((0, 0), (0, _NF - _NH)))
    xa = _tc_xa(x, w_top_pad)

    gA, pcA = _sc_gather(xa, c3a, r3a)
    gB, pcB = _sc_gather(xa, c3b, r3b)

    eargs = (W1a[_NF:], b1a.reshape(1, _NH), g1.reshape(1, _NH),
             be1.reshape(1, _NH), W1b, b1b.reshape(1, _NH))
    hpA = _tc_edge(gA, edge_attr, *eargs, 0)
    hpB = _tc_edge(gB, edge_attr, *eargs, _NE_H // _EB)

    psA = _sc_scatter(hpA, bia)
    psB = _sc_scatter(hpB, bib)

    n = _N_NODES
    return _tc_node(x, psA[:n], psA[n:], psB[:n], psB[n:],
                    pcA[:n], pcA[n:], pcB[:n], pcB[n:],
                    W2a[:_NF], W2a[_NF:],
                    b2a.reshape(1, _NH), g2.reshape(1, _NH),
                    be2.reshape(1, _NH), W2b, b2b.reshape(1, _NT))
